# baseline (speedup 1.0000x reference)
import jax
import jax.numpy as jnp
from jax import lax
from jax.experimental import pallas as pl
from jax.experimental.pallas import tpu as pltpu
from jax.experimental.pallas import tpu_sc as plsc

N_ROWS = 100000
W = 1024
D = 512
ROW_BLOCK = 4000
NC, NS = 2, 16


def _tc_scan_kernel(pattern_ref, proj_ref, stored_ref,
                    sim_ref, idx_ref,
                    cvec, hsum, best_val, best_idx):
    i = pl.program_id(0)

    @pl.when(i == 0)
    def _():
        projected = jax.lax.dot_general(
            pattern_ref[...], proj_ref[...],
            (((1,), (0,)), ((), ())),
            preferred_element_type=jnp.float32,
        )
        h = (projected > 0).astype(jnp.float32)
        cvec[...] = 2.0 * h - 1.0
        hsum[0] = jnp.sum(h)

    scores = jax.lax.dot_general(
        stored_ref[...], cvec[...],
        (((1,), (1,)), ((), ())),
        preferred_element_type=jnp.float32,
    )
    m = jnp.max(scores)
    rows = jax.lax.broadcasted_iota(jnp.int32, (ROW_BLOCK, 1), 0)
    local = jnp.min(jnp.where(scores == m, rows, N_ROWS))
    gidx = i * ROW_BLOCK + local

    @pl.when((i == 0) | (m > best_val[0]))
    def _():
        best_val[0] = m
        best_idx[0] = gidx

    @pl.when(i == pl.num_programs(0) - 1)
    def _():
        sim_ref[0, 0] = (W - hsum[0] + best_val[0]) / W
        idx_ref[0] = best_idx[0]


def _sc_gather_kernel(idx_hbm, pred_hbm, out_hbm, idx_v, row_v, sem):
    wid = lax.axis_index("s") * NC + lax.axis_index("c")

    @pl.when(wid == 0)
    def _():
        pltpu.sync_copy(idx_hbm, idx_v)
        pltpu.async_copy(pred_hbm.at[idx_v], row_v, sem).wait()
        pltpu.sync_copy(row_v, out_hbm)


def kernel(pattern, hash_projections, stored_hashes, predictions):
    best_sim, best_idx = pl.pallas_call(
        _tc_scan_kernel,
        grid=(N_ROWS // ROW_BLOCK,),
        out_shape=(
            jax.ShapeDtypeStruct((1, 1), jnp.float32),
            jax.ShapeDtypeStruct((1,), jnp.int32),
        ),
        in_specs=[
            pl.BlockSpec((1, D), lambda i: (0, 0)),
            pl.BlockSpec((D, W), lambda i: (0, 0)),
            pl.BlockSpec((ROW_BLOCK, W), lambda i: (i, 0)),
        ],
        out_specs=(
            pl.BlockSpec(memory_space=pltpu.SMEM),
            pl.BlockSpec(memory_space=pltpu.SMEM),
        ),
        scratch_shapes=[
            pltpu.VMEM((1, W), jnp.float32),
            pltpu.SMEM((1,), jnp.float32),
            pltpu.SMEM((1,), jnp.float32),
            pltpu.SMEM((1,), jnp.int32),
        ],
    )(pattern.reshape(1, D), hash_projections, stored_hashes)

    sc_gather = pl.kernel(
        _sc_gather_kernel,
        out_type=jax.ShapeDtypeStruct((1, D), jnp.float32),
        mesh=plsc.VectorSubcoreMesh(
            core_axis_name="c", subcore_axis_name="s",
            num_cores=NC, num_subcores=NS),
        compiler_params=pltpu.CompilerParams(needs_layout_passes=False),
        scratch_types=[
            pltpu.VMEM((1,), jnp.int32),
            pltpu.VMEM((1, D), jnp.float32),
            pltpu.SemaphoreType.DMA,
        ],
    )
    prediction = sc_gather(best_idx, predictions)

    return (prediction.reshape(D), best_sim.reshape(()), best_idx.reshape(()))
